# baseline (speedup 1.0000x reference)
import functools

import jax
import jax.numpy as jnp
from jax import lax
from jax.experimental import pallas as pl
from jax.experimental.pallas import tpu as pltpu
from jax.experimental.pallas import tpu_sc as plsc

NC = 2
NS = 16
L = 16

CH = 128
SPB = 1
BLK = CH * SPB
NBUF = 3
LOOK = 1

_DNUMS = lax.GatherDimensionNumbers(
    offset_dims=(), collapsed_slice_dims=(0,), start_index_map=(0,))


def _bcast_lane(v, j):
    idx = jnp.full((L, 1), j, jnp.int32)
    return lax.gather(v, idx, _DNUMS, slice_sizes=(1,),
                      mode=lax.GatherScatterMode.PROMISE_IN_BOUNDS)


def _matmul_body(x_ref, wt_ref, b_ref, o_ref):
    h = jnp.dot(x_ref[...], wt_ref[...],
                preferred_element_type=jnp.float32) + b_ref[...]
    dh = h.shape[-1] // 2
    o_ref[0] = h[:, :dh].astype(jnp.bfloat16)
    o_ref[1] = h[:, dh:].astype(jnp.bfloat16)


def _linear_split(x, W, b):
    n, d_in = x.shape
    d_out = W.shape[0]
    dh = d_out // 2
    bm = 2000
    return pl.pallas_call(
        _matmul_body,
        grid=(n // bm,),
        in_specs=[pl.BlockSpec((bm, d_in), lambda i: (i, 0)),
                  pl.BlockSpec((d_in, d_out), lambda i: (0, 0)),
                  pl.BlockSpec((1, d_out), lambda i: (0, 0))],
        out_specs=pl.BlockSpec((2, bm, dh), lambda i: (0, i, 0)),
        out_shape=jax.ShapeDtypeStruct((2, n, dh), jnp.bfloat16),
    )(x, W.T, b.reshape(1, d_out))


def _cat_body(p_ref, o_ref):
    dh = p_ref.shape[-1]
    o_ref[:, :dh] = p_ref[0]
    o_ref[:, dh:] = p_ref[1]


def _final_cat(p):
    _, n_pad, dh = p.shape
    bm = 2000
    assert n_pad % bm == 0
    return pl.pallas_call(
        _cat_body,
        grid=(n_pad // bm,),
        in_specs=[pl.BlockSpec((NC, bm, dh), lambda i: (0, i, 0))],
        out_specs=pl.BlockSpec((bm, NC * dh), lambda i: (i, 0)),
        out_shape=jax.ShapeDtypeStruct((n_pad, NC * dh), jnp.float32),
    )(p)


def _sc_body(n, dh, nblk, rows_per_sub, zchunks,
             h_hbm, col_hbm, row_hbm, w_hbm, out_hbm,
             col_v, row_v, w_v, rows_v, out_v, zbuf_v, acc_sh, *sems):
    gsems = sems[:NBUF]
    ssems = sems[NBUF:]
    cid = lax.axis_index("c")
    sid = lax.axis_index("s")

    zr = zchunks[0]
    @pl.loop(0, zr)
    def _(r):
        for f in range(dh // L):
            zbuf_v[r, pl.ds(f * L, L)] = jnp.zeros((L,), jnp.float32)

    zoff = 0
    for zc in zchunks:
        pltpu.sync_copy(
            zbuf_v.at[pl.ds(0, zc)],
            acc_sh.at[pl.ds(sid * rows_per_sub + zoff, zc)])
        zoff += zc
    plsc.subcore_barrier()

    nrows = SPB * nblk
    ibase = sid * nrows
    pltpu.sync_copy(col_hbm.at[pl.ds(ibase, nrows)], col_v)
    pltpu.sync_copy(row_hbm.at[pl.ds(ibase, nrows)], row_v)

    coff = (cid * n).astype(jnp.int32) * jnp.ones((L,), jnp.int32)
    @pl.loop(0, nrows)
    def _(r):
        for f in range(CH // L):
            sl = pl.ds(f * L, L)
            col_v[r, sl] = col_v[r, sl] + coff

    def g_issue(h, b):
        for s in range(SPB):
            pltpu.async_copy(h_hbm.at[col_v.at[h * SPB + s]],
                             rows_v.at[pl.ds(b * BLK + s * CH, CH)], gsems[b])
        pltpu.async_copy(w_hbm.at[pl.ds(ibase + h, 1)],
                         w_v.at[pl.ds(b, 1)], gsems[b])

    def g_wait(h, b):
        for s in range(SPB):
            pltpu.make_async_copy(
                h_hbm.at[col_v.at[h * SPB + s]],
                rows_v.at[pl.ds(b * BLK + s * CH, CH)], gsems[b]).wait()
        pltpu.make_async_copy(w_hbm.at[pl.ds(ibase + h, 1)],
                              w_v.at[pl.ds(b, 1)], gsems[b]).wait()

    def s_issue(h, b):
        for s in range(SPB):
            pltpu.async_copy(out_v.at[pl.ds(b * BLK + s * CH, CH)],
                             acc_sh.at[row_v.at[h * SPB + s]], ssems[b],
                             add=True)

    def s_wait(h, b):
        for s in range(SPB):
            pltpu.make_async_copy(
                out_v.at[pl.ds(b * BLK + s * CH, CH)],
                acc_sh.at[row_v.at[h * SPB + s]], ssems[b]).wait()

    mask_hi = jnp.full((L,), -65536, jnp.int32)

    def compute(h, b):
        @pl.loop(0, BLK // L)
        def _(g):
            w16 = w_v[b, pl.ds(g * L, L)]
            r = b * BLK + g * L
            for j in range(L):
                wb = _bcast_lane(w16, j)
                for q in range(dh // (2 * L)):
                    raw = rows_v[r + j, pl.ds(q * 2 * L, 2 * L)]
                    w32 = plsc.bitcast(raw, jnp.int32)
                    lo = lax.bitcast_convert_type(
                        lax.shift_left(w32, 16), jnp.float32)
                    hi = lax.bitcast_convert_type(w32 & mask_hi, jnp.float32)
                    out_v[r + j, pl.ds(q * 2 * L, L)] = lo * wb
                    out_v[r + j, pl.ds(q * 2 * L + L, L)] = hi * wb

    for p in range(LOOK):
        g_issue(p, p)

    @pl.loop(0, nblk // NBUF)
    def _(rr):
        for b in range(NBUF):
            h = rr * NBUF + b
            nxt = (b + LOOK) % NBUF

            @pl.when(h >= NBUF - LOOK)
            def _():
                s_wait(h - (NBUF - LOOK), nxt)

            @pl.when(h + LOOK < nblk)
            def _():
                g_issue(h + LOOK, nxt)

            g_wait(h, b)
            compute(h, b)
            s_issue(h, b)

    for t in range(NBUF - LOOK):
        s_wait(nblk - (NBUF - LOOK) + t, (nblk - (NBUF - LOOK) + t) % NBUF)

    plsc.subcore_barrier()
    r0 = sid * rows_per_sub
    pltpu.sync_copy(acc_sh.at[pl.ds(r0, rows_per_sub)],
                    out_hbm.at[cid, pl.ds(r0, rows_per_sub)])


def _sc_scatter(h2, col_p, row_p, w_p, nblk, n, n_pad):
    dh = h2.shape[-1]
    h_flat = h2.reshape(NC * n, dh)
    rows_per_sub = n_pad // NS
    zchunks = []
    left = rows_per_sub
    while left > 0:
        zc = min(80, left)
        zchunks.append(zc)
        left -= zc
    mesh = plsc.VectorSubcoreMesh(core_axis_name="c", subcore_axis_name="s",
                                  num_cores=NC)
    body = functools.partial(_sc_body, n, dh, nblk, rows_per_sub,
                             tuple(zchunks))
    return pl.kernel(
        body,
        out_type=pltpu.HBM((NC, n_pad, dh), jnp.float32),
        mesh=mesh,
        compiler_params=pltpu.CompilerParams(use_tc_tiling_on_sc=False,
                                             needs_layout_passes=False),
        scratch_types=[
            pltpu.VMEM((SPB * nblk, CH), jnp.int32),
            pltpu.VMEM((SPB * nblk, CH), jnp.int32),
            pltpu.VMEM((NBUF, CH), jnp.float32),
            pltpu.VMEM((NBUF * BLK, dh), jnp.bfloat16),
            pltpu.VMEM((NBUF * BLK, dh), jnp.float32),
            pltpu.VMEM((zchunks[0], dh), jnp.float32),
            pltpu.VMEM_SHARED((n_pad, dh), jnp.float32),
        ] + [pltpu.SemaphoreType.DMA] * (2 * NBUF),
    )(h_flat, col_p, row_p, w_p)


def _pair_perm(d_out):
    perm = []
    for g in range(d_out // 32):
        for i in range(L):
            perm.append(32 * g + i)
            perm.append(32 * g + L + i)
    return perm


def kernel(x, edge_index, edge_weight, W, b):
    n = x.shape[0]
    e = edge_index.shape[1]
    row = edge_index[0].astype(jnp.int32)
    col = edge_index[1].astype(jnp.int32)
    w = edge_weight.astype(jnp.float32)

    perm = jnp.asarray(_pair_perm(W.shape[0]))
    W = W[perm]
    b = b[perm]

    per_s = -(-e // (NS * BLK * NBUF)) * (BLK * NBUF)
    e_pad = per_s * NS
    pad = e_pad - e
    row_p = jnp.concatenate([row, jnp.zeros((pad,), jnp.int32)])
    col_p = jnp.concatenate([col, jnp.zeros((pad,), jnp.int32)])
    w_p = jnp.concatenate([w, jnp.zeros((pad,), jnp.float32)])
    shape2d = (e_pad // CH, CH)

    n_pad = n

    h2 = _linear_split(x, W, b)
    partials = _sc_scatter(h2, col_p.reshape(shape2d), row_p.reshape(shape2d),
                           w_p.reshape(shape2d), per_s // BLK, n, n_pad)
    return _final_cat(partials)


# speedup vs baseline: 1.4784x; 1.4784x over previous
import functools

import jax
import jax.numpy as jnp
from jax import lax
from jax.experimental import pallas as pl
from jax.experimental.pallas import tpu as pltpu
from jax.experimental.pallas import tpu_sc as plsc

NC = 2
NS = 16
L = 16

CH = 128
SPB = 1
BLK = CH * SPB
NBUF = 3
LOOK = 1

_DNUMS = lax.GatherDimensionNumbers(
    offset_dims=(), collapsed_slice_dims=(0,), start_index_map=(0,))


def _bcast_lane(v, j):
    idx = jnp.full((L, 1), j, jnp.int32)
    return lax.gather(v, idx, _DNUMS, slice_sizes=(1,),
                      mode=lax.GatherScatterMode.PROMISE_IN_BOUNDS)


def _matmul_body(x_ref, wt_ref, b_ref, o_ref):
    h = jnp.dot(x_ref[...], wt_ref[...],
                preferred_element_type=jnp.float32) + b_ref[...]
    dh = h.shape[-1] // 2
    o_ref[0] = h[:, :dh].astype(jnp.bfloat16)
    o_ref[1] = h[:, dh:].astype(jnp.bfloat16)


def _linear_split(x, W, b):
    n, d_in = x.shape
    d_out = W.shape[0]
    dh = d_out // 2
    bm = 2000
    return pl.pallas_call(
        _matmul_body,
        grid=(n // bm,),
        in_specs=[pl.BlockSpec((bm, d_in), lambda i: (i, 0)),
                  pl.BlockSpec((d_in, d_out), lambda i: (0, 0)),
                  pl.BlockSpec((1, d_out), lambda i: (0, 0))],
        out_specs=pl.BlockSpec((2, bm, dh), lambda i: (0, i, 0)),
        out_shape=jax.ShapeDtypeStruct((2, n, dh), jnp.bfloat16),
    )(x, W.T, b.reshape(1, d_out))


def _cat_body(p_ref, pm_ref, o_ref):
    dh = p_ref.shape[-1]
    o_ref[:, :dh] = jnp.dot(p_ref[0], pm_ref[...],
                            preferred_element_type=jnp.float32)
    o_ref[:, dh:] = jnp.dot(p_ref[1], pm_ref[...],
                            preferred_element_type=jnp.float32)


def _final_cat(p):
    _, n_pad, dh = p.shape
    bm = 2000
    assert n_pad % bm == 0
    pm = jnp.zeros((dh, dh), jnp.bfloat16).at[
        jnp.arange(dh), jnp.asarray(_pair_perm(dh))].set(1)
    return pl.pallas_call(
        _cat_body,
        grid=(n_pad // bm,),
        in_specs=[pl.BlockSpec((NC, bm, dh), lambda i: (0, i, 0)),
                  pl.BlockSpec((dh, dh), lambda i: (0, 0))],
        out_specs=pl.BlockSpec((bm, NC * dh), lambda i: (i, 0)),
        out_shape=jax.ShapeDtypeStruct((n_pad, NC * dh), jnp.float32),
    )(p, pm)


def _sc_body(n, dh, nblk, rows_per_sub, zchunks,
             h_hbm, col_hbm, row_hbm, w_hbm, out_hbm,
             col_v, row_v, w_v, rows_v, out_v, zbuf_v, acc_sh, *sems):
    gsems = sems[:NBUF]
    ssems = sems[NBUF:]
    cid = lax.axis_index("c")
    sid = lax.axis_index("s")

    zr = zchunks[0]
    @pl.loop(0, zr)
    def _(r):
        for f in range(dh // (2 * L)):
            zbuf_v[r, pl.ds(f * 2 * L, 2 * L)] = jnp.zeros(
                (2 * L,), jnp.bfloat16)

    zoff = 0
    for zc in zchunks:
        pltpu.sync_copy(
            zbuf_v.at[pl.ds(0, zc)],
            acc_sh.at[pl.ds(sid * rows_per_sub + zoff, zc)])
        zoff += zc
    plsc.subcore_barrier()

    nrows = SPB * nblk
    ibase = sid * nrows
    pltpu.sync_copy(col_hbm.at[pl.ds(ibase, nrows)], col_v)
    pltpu.sync_copy(row_hbm.at[pl.ds(ibase, nrows)], row_v)

    coff = (cid * n).astype(jnp.int32) * jnp.ones((L,), jnp.int32)
    @pl.loop(0, nrows)
    def _(r):
        for f in range(CH // L):
            sl = pl.ds(f * L, L)
            col_v[r, sl] = col_v[r, sl] + coff

    def g_issue(h, b):
        for s in range(SPB):
            pltpu.async_copy(h_hbm.at[col_v.at[h * SPB + s]],
                             rows_v.at[pl.ds(b * BLK + s * CH, CH)], gsems[b])
        pltpu.async_copy(w_hbm.at[pl.ds(ibase + h, 1)],
                         w_v.at[pl.ds(b, 1)], gsems[b])

    def g_wait(h, b):
        for s in range(SPB):
            pltpu.make_async_copy(
                h_hbm.at[col_v.at[h * SPB + s]],
                rows_v.at[pl.ds(b * BLK + s * CH, CH)], gsems[b]).wait()
        pltpu.make_async_copy(w_hbm.at[pl.ds(ibase + h, 1)],
                              w_v.at[pl.ds(b, 1)], gsems[b]).wait()

    def s_issue(h, b):
        for s in range(SPB):
            pltpu.async_copy(out_v.at[pl.ds(b * BLK + s * CH, CH)],
                             acc_sh.at[row_v.at[h * SPB + s]], ssems[b],
                             add=True)

    def s_wait(h, b):
        for s in range(SPB):
            pltpu.make_async_copy(
                out_v.at[pl.ds(b * BLK + s * CH, CH)],
                acc_sh.at[row_v.at[h * SPB + s]], ssems[b]).wait()

    mask_hi = jnp.full((L,), -65536, jnp.int32)

    def compute(h, b):
        @pl.loop(0, BLK // L)
        def _(g):
            w16 = w_v[b, pl.ds(g * L, L)]
            r = b * BLK + g * L
            for j in range(L):
                wb = _bcast_lane(w16, j)
                for q in range(dh // (2 * L)):
                    raw = rows_v[r + j, pl.ds(q * 2 * L, 2 * L)]
                    w32 = plsc.bitcast(raw, jnp.int32)
                    lo = lax.bitcast_convert_type(
                        lax.shift_left(w32, 16), jnp.float32)
                    hi = lax.bitcast_convert_type(w32 & mask_hi, jnp.float32)
                    out_v[r + j, pl.ds(q * 2 * L, 2 * L)] = plsc.pack(
                        lo * wb, hi * wb, format=plsc.PackFormat.INTERLEAVED)

    for p in range(LOOK):
        g_issue(p, p)

    @pl.loop(0, nblk // NBUF)
    def _(rr):
        for b in range(NBUF):
            h = rr * NBUF + b
            nxt = (b + LOOK) % NBUF

            @pl.when(h >= NBUF - LOOK)
            def _():
                s_wait(h - (NBUF - LOOK), nxt)

            @pl.when(h + LOOK < nblk)
            def _():
                g_issue(h + LOOK, nxt)

            g_wait(h, b)
            compute(h, b)
            s_issue(h, b)

    for t in range(NBUF - LOOK):
        s_wait(nblk - (NBUF - LOOK) + t, (nblk - (NBUF - LOOK) + t) % NBUF)

    plsc.subcore_barrier()
    r0 = sid * rows_per_sub
    pltpu.sync_copy(acc_sh.at[pl.ds(r0, rows_per_sub)],
                    out_hbm.at[cid, pl.ds(r0, rows_per_sub)])


def _sc_scatter(h2, col_p, row_p, w_p, nblk, n, n_pad):
    dh = h2.shape[-1]
    h_flat = h2.reshape(NC * n, dh)
    rows_per_sub = n_pad // NS
    zchunks = []
    left = rows_per_sub
    while left > 0:
        zc = min(80, left)
        zchunks.append(zc)
        left -= zc
    mesh = plsc.VectorSubcoreMesh(core_axis_name="c", subcore_axis_name="s",
                                  num_cores=NC)
    body = functools.partial(_sc_body, n, dh, nblk, rows_per_sub,
                             tuple(zchunks))
    return pl.kernel(
        body,
        out_type=pltpu.HBM((NC, n_pad, dh), jnp.bfloat16),
        mesh=mesh,
        compiler_params=pltpu.CompilerParams(use_tc_tiling_on_sc=False,
                                             needs_layout_passes=False),
        scratch_types=[
            pltpu.VMEM((SPB * nblk, CH), jnp.int32),
            pltpu.VMEM((SPB * nblk, CH), jnp.int32),
            pltpu.VMEM((NBUF, CH), jnp.float32),
            pltpu.VMEM((NBUF * BLK, dh), jnp.bfloat16),
            pltpu.VMEM((NBUF * BLK, dh), jnp.bfloat16),
            pltpu.VMEM((zchunks[0], dh), jnp.bfloat16),
            pltpu.VMEM_SHARED((n_pad, dh), jnp.bfloat16),
        ] + [pltpu.SemaphoreType.DMA] * (2 * NBUF),
    )(h_flat, col_p, row_p, w_p)


def _pair_perm(d_out):
    perm = []
    for g in range(d_out // 32):
        for i in range(L):
            perm.append(32 * g + i)
            perm.append(32 * g + L + i)
    return perm


def kernel(x, edge_index, edge_weight, W, b):
    n = x.shape[0]
    e = edge_index.shape[1]
    row = edge_index[0].astype(jnp.int32)
    col = edge_index[1].astype(jnp.int32)
    w = edge_weight.astype(jnp.float32)

    perm = jnp.asarray(_pair_perm(W.shape[0]))
    W = W[perm]
    b = b[perm]

    per_s = -(-e // (NS * BLK * NBUF)) * (BLK * NBUF)
    e_pad = per_s * NS
    pad = e_pad - e
    row_p = jnp.concatenate([row, jnp.zeros((pad,), jnp.int32)])
    col_p = jnp.concatenate([col, jnp.zeros((pad,), jnp.int32)])
    w_p = jnp.concatenate([w, jnp.zeros((pad,), jnp.float32)])
    shape2d = (e_pad // CH, CH)

    n_pad = n

    h2 = _linear_split(x, W, b)
    partials = _sc_scatter(h2, col_p.reshape(shape2d), row_p.reshape(shape2d),
                           w_p.reshape(shape2d), per_s // BLK, n, n_pad)
    return _final_cat(partials)


# speedup vs baseline: 2.0723x; 1.4017x over previous
import functools

import jax
import jax.numpy as jnp
from jax import lax
from jax.experimental import pallas as pl
from jax.experimental.pallas import tpu as pltpu
from jax.experimental.pallas import tpu_sc as plsc

NC = 2
NS = 16
L = 16

CH = 128
SPB = 1
BLK = CH * SPB
NBUF = 3
EBUF = 6

_DNUMS = lax.GatherDimensionNumbers(
    offset_dims=(), collapsed_slice_dims=(0,), start_index_map=(0,))


def _bcast_lane(v, j):
    idx = jnp.full((L, 1), j, jnp.int32)
    return lax.gather(v, idx, _DNUMS, slice_sizes=(1,),
                      mode=lax.GatherScatterMode.PROMISE_IN_BOUNDS)


def _matmul_body(x_ref, wt_ref, b_ref, o_ref):
    h = jnp.dot(x_ref[...], wt_ref[...],
                preferred_element_type=jnp.float32) + b_ref[...]
    dh = h.shape[-1] // 2
    o_ref[0] = h[:, :dh].astype(jnp.bfloat16)
    o_ref[1] = h[:, dh:].astype(jnp.bfloat16)


def _linear_split(x, W, b):
    n, d_in = x.shape
    d_out = W.shape[0]
    dh = d_out // 2
    bm = 2000
    return pl.pallas_call(
        _matmul_body,
        grid=(n // bm,),
        in_specs=[pl.BlockSpec((bm, d_in), lambda i: (i, 0)),
                  pl.BlockSpec((d_in, d_out), lambda i: (0, 0)),
                  pl.BlockSpec((1, d_out), lambda i: (0, 0))],
        out_specs=pl.BlockSpec((2, bm, dh), lambda i: (0, i, 0)),
        out_shape=jax.ShapeDtypeStruct((2, n, dh), jnp.bfloat16),
    )(x, W.T, b.reshape(1, d_out))


def _cat_body(p_ref, pm_ref, o_ref):
    dh = p_ref.shape[-1]
    o_ref[:, :dh] = jnp.dot(p_ref[0], pm_ref[...],
                            preferred_element_type=jnp.float32)
    o_ref[:, dh:] = jnp.dot(p_ref[1], pm_ref[...],
                            preferred_element_type=jnp.float32)


def _final_cat(p):
    _, n_pad, dh = p.shape
    bm = 2000
    assert n_pad % bm == 0
    pm = jnp.zeros((dh, dh), jnp.bfloat16).at[
        jnp.arange(dh), jnp.asarray(_pair_perm(dh))].set(1)
    return pl.pallas_call(
        _cat_body,
        grid=(n_pad // bm,),
        in_specs=[pl.BlockSpec((NC, bm, dh), lambda i: (0, i, 0)),
                  pl.BlockSpec((dh, dh), lambda i: (0, 0))],
        out_specs=pl.BlockSpec((bm, NC * dh), lambda i: (i, 0)),
        out_shape=jax.ShapeDtypeStruct((n_pad, NC * dh), jnp.float32),
    )(p, pm)


def _sc_body(n, dh, nblk, rows_per_sub, zchunks,
             h_hbm, edata_hbm, out_hbm,
             e_v, rows_v, out_v, zbuf_v, acc_sh, hc_sh, *sems):
    gsems = sems[:NBUF]
    ssems = sems[NBUF:2 * NBUF]
    esems = sems[2 * NBUF:]
    cid = lax.axis_index("c")
    sid = lax.axis_index("s")

    zr = zchunks[0]
    @pl.loop(0, zr)
    def _(r):
        for f in range(dh // (2 * L)):
            zbuf_v[r, pl.ds(f * 2 * L, 2 * L)] = jnp.zeros(
                (2 * L,), jnp.bfloat16)

    zoff = 0
    for zc in zchunks:
        pltpu.sync_copy(
            zbuf_v.at[pl.ds(0, zc)],
            acc_sh.at[pl.ds(sid * rows_per_sub + zoff, zc)])
        zoff += zc

    hrows = n // NS
    pltpu.sync_copy(h_hbm.at[pl.ds(cid * n + sid * hrows, hrows)],
                    hc_sh.at[pl.ds(sid * hrows, hrows)])
    plsc.subcore_barrier()

    eblock0 = sid * nblk

    def i_issue(h, eb):
        pltpu.async_copy(edata_hbm.at[pl.ds((eblock0 + h) * 3, 3)],
                         e_v.at[pl.ds(eb * 3, 3)], esems[eb])

    def i_wait(h, eb):
        pltpu.make_async_copy(edata_hbm.at[pl.ds((eblock0 + h) * 3, 3)],
                              e_v.at[pl.ds(eb * 3, 3)], esems[eb]).wait()

    def g_issue(h, b, eb):
        pltpu.async_copy(hc_sh.at[e_v.at[eb * 3]],
                         rows_v.at[pl.ds(b * BLK, CH)], gsems[b])

    def g_wait(h, b, eb):
        pltpu.make_async_copy(hc_sh.at[e_v.at[eb * 3]],
                              rows_v.at[pl.ds(b * BLK, CH)], gsems[b]).wait()

    def s_issue(h, b, eb):
        pltpu.async_copy(out_v.at[pl.ds(b * BLK, CH)],
                         acc_sh.at[e_v.at[eb * 3 + 1]], ssems[b], add=True)

    def s_wait(h, b, eb):
        pltpu.make_async_copy(out_v.at[pl.ds(b * BLK, CH)],
                              acc_sh.at[e_v.at[eb * 3 + 1]], ssems[b]).wait()

    mask_hi = jnp.full((L,), -65536, jnp.int32)

    def compute(h, b, eb):
        @pl.loop(0, BLK // L)
        def _(g):
            w16 = plsc.bitcast(e_v[eb * 3 + 2, pl.ds(g * L, L)], jnp.float32)
            r = b * BLK + g * L
            for j in range(L):
                wb = _bcast_lane(w16, j)
                for q in range(dh // (2 * L)):
                    raw = rows_v[r + j, pl.ds(q * 2 * L, 2 * L)]
                    w32 = plsc.bitcast(raw, jnp.int32)
                    lo = lax.bitcast_convert_type(
                        lax.shift_left(w32, 16), jnp.float32)
                    hi = lax.bitcast_convert_type(w32 & mask_hi, jnp.float32)
                    out_v[r + j, pl.ds(q * 2 * L, 2 * L)] = plsc.pack(
                        lo * wb, hi * wb, format=plsc.PackFormat.INTERLEAVED)

    i_issue(0, 0)
    i_issue(1, 1)
    i_wait(0, 0)
    g_issue(0, 0, 0)

    @pl.loop(0, nblk // EBUF)
    def _(rr):
        for b in range(EBUF):
            h = rr * EBUF + b
            d = b % NBUF
            nd = (b + 1) % NBUF
            ne = (b + 1) % EBUF
            n2e = (b + 2) % EBUF

            @pl.when(h >= 2)
            def _():
                s_wait(h - 2, nd, (b - 2) % EBUF)

            @pl.when(h + 2 < nblk)
            def _():
                i_issue(h + 2, n2e)

            @pl.when(h + 1 < nblk)
            def _():
                i_wait(h + 1, ne)
                g_issue(h + 1, nd, ne)

            g_wait(h, d, b)
            compute(h, d, b)
            s_issue(h, d, b)

    s_wait(nblk - 2, (nblk - 2) % NBUF, (nblk - 2) % EBUF)
    s_wait(nblk - 1, (nblk - 1) % NBUF, (nblk - 1) % EBUF)

    plsc.subcore_barrier()
    r0 = sid * rows_per_sub
    pltpu.sync_copy(acc_sh.at[pl.ds(r0, rows_per_sub)],
                    out_hbm.at[cid, pl.ds(r0, rows_per_sub)])


def _sc_scatter(h2, edata, nblk, n, n_pad):
    dh = h2.shape[-1]
    h_flat = h2.reshape(NC * n, dh)
    rows_per_sub = n_pad // NS
    zchunks = []
    left = rows_per_sub
    while left > 0:
        zc = min(80, left)
        zchunks.append(zc)
        left -= zc
    mesh = plsc.VectorSubcoreMesh(core_axis_name="c", subcore_axis_name="s",
                                  num_cores=NC)
    body = functools.partial(_sc_body, n, dh, nblk, rows_per_sub,
                             tuple(zchunks))
    return pl.kernel(
        body,
        out_type=pltpu.HBM((NC, n_pad, dh), jnp.bfloat16),
        mesh=mesh,
        compiler_params=pltpu.CompilerParams(use_tc_tiling_on_sc=False,
                                             needs_layout_passes=False),
        scratch_types=[
            pltpu.VMEM((EBUF * 3, CH), jnp.int32),
            pltpu.VMEM((NBUF * BLK, dh), jnp.bfloat16),
            pltpu.VMEM((NBUF * BLK, dh), jnp.bfloat16),
            pltpu.VMEM((zchunks[0], dh), jnp.bfloat16),
            pltpu.VMEM_SHARED((n_pad, dh), jnp.bfloat16),
            pltpu.VMEM_SHARED((n, dh), jnp.bfloat16),
        ] + [pltpu.SemaphoreType.DMA] * (2 * NBUF + EBUF),
    )(h_flat, edata)


def _pair_perm(d_out):
    perm = []
    for g in range(d_out // 32):
        for i in range(L):
            perm.append(32 * g + i)
            perm.append(32 * g + L + i)
    return perm


def kernel(x, edge_index, edge_weight, W, b):
    n = x.shape[0]
    e = edge_index.shape[1]
    row = edge_index[0].astype(jnp.int32)
    col = edge_index[1].astype(jnp.int32)
    w = edge_weight.astype(jnp.float32)

    perm = jnp.asarray(_pair_perm(W.shape[0]))
    W = W[perm]
    b = b[perm]

    per_s = -(-e // (NS * BLK * EBUF)) * (BLK * EBUF)
    e_pad = per_s * NS
    pad = e_pad - e
    row_p = jnp.concatenate([row, jnp.zeros((pad,), jnp.int32)])
    col_p = jnp.concatenate([col, jnp.zeros((pad,), jnp.int32)])
    w_p = jnp.concatenate([w, jnp.zeros((pad,), jnp.float32)])
    nblocks = e_pad // CH
    edata = jnp.stack([col_p.reshape(nblocks, CH),
                       row_p.reshape(nblocks, CH),
                       lax.bitcast_convert_type(w_p, jnp.int32)
                       .reshape(nblocks, CH)], axis=1).reshape(3 * nblocks, CH)

    n_pad = n

    h2 = _linear_split(x, W, b)
    partials = _sc_scatter(h2, edata, per_s // BLK, n, n_pad)
    return _final_cat(partials)


# speedup vs baseline: 2.0823x; 1.0048x over previous
import functools

import jax
import jax.numpy as jnp
from jax import lax
from jax.experimental import pallas as pl
from jax.experimental.pallas import tpu as pltpu
from jax.experimental.pallas import tpu_sc as plsc

NC = 2
NS = 16
L = 16

CH = 128
SPB = 1
BLK = CH * SPB
NBUF = 3
EBUF = 6

_DNUMS = lax.GatherDimensionNumbers(
    offset_dims=(), collapsed_slice_dims=(0,), start_index_map=(0,))


def _bcast_lane(v, j):
    idx = jnp.full((L, 1), j, jnp.int32)
    return lax.gather(v, idx, _DNUMS, slice_sizes=(1,),
                      mode=lax.GatherScatterMode.PROMISE_IN_BOUNDS)


def _matmul_body(x_ref, wt_ref, b_ref, o_ref):
    h = jnp.dot(x_ref[...], wt_ref[...],
                preferred_element_type=jnp.float32) + b_ref[...]
    dh = h.shape[-1] // 2
    o_ref[0] = h[:, :dh].astype(jnp.bfloat16)
    o_ref[1] = h[:, dh:].astype(jnp.bfloat16)


def _linear_split(x, W, b):
    n, d_in = x.shape
    d_out = W.shape[0]
    dh = d_out // 2
    bm = 2000
    return pl.pallas_call(
        _matmul_body,
        grid=(n // bm,),
        in_specs=[pl.BlockSpec((bm, d_in), lambda i: (i, 0)),
                  pl.BlockSpec((d_in, d_out), lambda i: (0, 0)),
                  pl.BlockSpec((1, d_out), lambda i: (0, 0))],
        out_specs=pl.BlockSpec((2, bm, dh), lambda i: (0, i, 0)),
        out_shape=jax.ShapeDtypeStruct((2, n, dh), jnp.bfloat16),
    )(x, W.T, b.reshape(1, d_out))


def _cat_body(p_ref, pm_ref, o_ref):
    dh = p_ref.shape[-1]
    o_ref[:, :dh] = jnp.dot(p_ref[0], pm_ref[...],
                            preferred_element_type=jnp.float32)
    o_ref[:, dh:] = jnp.dot(p_ref[1], pm_ref[...],
                            preferred_element_type=jnp.float32)


def _final_cat(p):
    _, n_pad, dh = p.shape
    bm = 2000
    assert n_pad % bm == 0
    pm = jnp.zeros((dh, dh), jnp.bfloat16).at[
        jnp.arange(dh), jnp.asarray(_pair_perm(dh))].set(1)
    return pl.pallas_call(
        _cat_body,
        grid=(n_pad // bm,),
        in_specs=[pl.BlockSpec((NC, bm, dh), lambda i: (0, i, 0)),
                  pl.BlockSpec((dh, dh), lambda i: (0, 0))],
        out_specs=pl.BlockSpec((bm, NC * dh), lambda i: (i, 0)),
        out_shape=jax.ShapeDtypeStruct((n_pad, NC * dh), jnp.float32),
    )(p, pm)


def _sc_body(n, dh, nblk, rows_per_sub, zchunks,
             h_hbm, edata_hbm, out_hbm,
             e_v, rows_v, out_v, zbuf_v, acc_sh, hc_sh, *sems):
    gsems = sems[:NBUF]
    ssems = sems[NBUF:2 * NBUF]
    esems = sems[2 * NBUF:]
    cid = lax.axis_index("c")
    sid = lax.axis_index("s")

    zr = zchunks[0]
    @pl.loop(0, zr)
    def _(r):
        for f in range(dh // (2 * L)):
            zbuf_v[r, pl.ds(f * 2 * L, 2 * L)] = jnp.zeros(
                (2 * L,), jnp.bfloat16)

    zoff = 0
    for zc in zchunks:
        pltpu.sync_copy(
            zbuf_v.at[pl.ds(0, zc)],
            acc_sh.at[pl.ds(sid * rows_per_sub + zoff, zc)])
        zoff += zc

    hrows = n // NS
    pltpu.sync_copy(h_hbm.at[pl.ds(cid * n + sid * hrows, hrows)],
                    hc_sh.at[pl.ds(sid * hrows, hrows)])
    plsc.subcore_barrier()

    eblock0 = sid * nblk

    def i_issue(h, eb):
        pltpu.async_copy(edata_hbm.at[pl.ds((eblock0 + h) * 3, 3)],
                         e_v.at[pl.ds(eb * 3, 3)], esems[eb])

    def i_wait(h, eb):
        pltpu.make_async_copy(edata_hbm.at[pl.ds((eblock0 + h) * 3, 3)],
                              e_v.at[pl.ds(eb * 3, 3)], esems[eb]).wait()

    def g_issue(h, b, eb):
        pltpu.async_copy(hc_sh.at[e_v.at[eb * 3]],
                         rows_v.at[pl.ds(b * BLK, CH)], gsems[b])

    def g_wait(h, b, eb):
        pltpu.make_async_copy(hc_sh.at[e_v.at[eb * 3]],
                              rows_v.at[pl.ds(b * BLK, CH)], gsems[b]).wait()

    def s_issue(h, b, eb):
        pltpu.async_copy(out_v.at[pl.ds(b * BLK, CH)],
                         acc_sh.at[e_v.at[eb * 3 + 1]], ssems[b], add=True)

    def s_wait(h, b, eb):
        pltpu.make_async_copy(out_v.at[pl.ds(b * BLK, CH)],
                              acc_sh.at[e_v.at[eb * 3 + 1]], ssems[b]).wait()

    def compute(h, b, eb):
        @pl.loop(0, BLK // L)
        def _(g):
            w16 = plsc.bitcast(e_v[eb * 3 + 2, pl.ds(g * L, L)], jnp.float32)
            r = b * BLK + g * L
            for j in range(L):
                wb = _bcast_lane(w16, j)
                wb2 = plsc.pack(wb, wb, format=plsc.PackFormat.INTERLEAVED)
                for q in range(dh // (2 * L)):
                    sl = pl.ds(q * 2 * L, 2 * L)
                    out_v[r + j, sl] = rows_v[r + j, sl] * wb2

    i_issue(0, 0)
    i_issue(1, 1)
    i_wait(0, 0)
    g_issue(0, 0, 0)

    @pl.loop(0, nblk // EBUF)
    def _(rr):
        for b in range(EBUF):
            h = rr * EBUF + b
            d = b % NBUF
            nd = (b + 1) % NBUF
            ne = (b + 1) % EBUF
            n2e = (b + 2) % EBUF

            @pl.when(h >= 2)
            def _():
                s_wait(h - 2, nd, (b - 2) % EBUF)

            @pl.when(h + 2 < nblk)
            def _():
                i_issue(h + 2, n2e)

            @pl.when(h + 1 < nblk)
            def _():
                i_wait(h + 1, ne)
                g_issue(h + 1, nd, ne)

            g_wait(h, d, b)
            compute(h, d, b)
            s_issue(h, d, b)

    s_wait(nblk - 2, (nblk - 2) % NBUF, (nblk - 2) % EBUF)
    s_wait(nblk - 1, (nblk - 1) % NBUF, (nblk - 1) % EBUF)

    plsc.subcore_barrier()
    r0 = sid * rows_per_sub
    pltpu.sync_copy(acc_sh.at[pl.ds(r0, rows_per_sub)],
                    out_hbm.at[cid, pl.ds(r0, rows_per_sub)])


def _sc_scatter(h2, edata, nblk, n, n_pad):
    dh = h2.shape[-1]
    h_flat = h2.reshape(NC * n, dh)
    rows_per_sub = n_pad // NS
    zchunks = []
    left = rows_per_sub
    while left > 0:
        zc = min(80, left)
        zchunks.append(zc)
        left -= zc
    mesh = plsc.VectorSubcoreMesh(core_axis_name="c", subcore_axis_name="s",
                                  num_cores=NC)
    body = functools.partial(_sc_body, n, dh, nblk, rows_per_sub,
                             tuple(zchunks))
    return pl.kernel(
        body,
        out_type=pltpu.HBM((NC, n_pad, dh), jnp.bfloat16),
        mesh=mesh,
        compiler_params=pltpu.CompilerParams(use_tc_tiling_on_sc=False,
                                             needs_layout_passes=False),
        scratch_types=[
            pltpu.VMEM((EBUF * 3, CH), jnp.int32),
            pltpu.VMEM((NBUF * BLK, dh), jnp.bfloat16),
            pltpu.VMEM((NBUF * BLK, dh), jnp.bfloat16),
            pltpu.VMEM((zchunks[0], dh), jnp.bfloat16),
            pltpu.VMEM_SHARED((n_pad, dh), jnp.bfloat16),
            pltpu.VMEM_SHARED((n, dh), jnp.bfloat16),
        ] + [pltpu.SemaphoreType.DMA] * (2 * NBUF + EBUF),
    )(h_flat, edata)


def _pair_perm(d_out):
    perm = []
    for g in range(d_out // 32):
        for i in range(L):
            perm.append(32 * g + i)
            perm.append(32 * g + L + i)
    return perm


def kernel(x, edge_index, edge_weight, W, b):
    n = x.shape[0]
    e = edge_index.shape[1]
    row = edge_index[0].astype(jnp.int32)
    col = edge_index[1].astype(jnp.int32)
    w = edge_weight.astype(jnp.float32)

    perm = jnp.asarray(_pair_perm(W.shape[0]))
    W = W[perm]
    b = b[perm]

    per_s = -(-e // (NS * BLK * EBUF)) * (BLK * EBUF)
    e_pad = per_s * NS
    pad = e_pad - e
    row_p = jnp.concatenate([row, jnp.zeros((pad,), jnp.int32)])
    col_p = jnp.concatenate([col, jnp.zeros((pad,), jnp.int32)])
    w_p = jnp.concatenate([w, jnp.zeros((pad,), jnp.float32)])
    nblocks = e_pad // CH
    edata = jnp.stack([col_p.reshape(nblocks, CH),
                       row_p.reshape(nblocks, CH),
                       lax.bitcast_convert_type(w_p, jnp.int32)
                       .reshape(nblocks, CH)], axis=1).reshape(3 * nblocks, CH)

    n_pad = n

    h2 = _linear_split(x, W, b)
    partials = _sc_scatter(h2, edata, per_s // BLK, n, n_pad)
    return _final_cat(partials)
